# baseline (speedup 1.0000x reference)
import functools

import jax
import jax.numpy as jnp
from jax import lax
from jax.experimental import pallas as pl
from jax.experimental.pallas import tpu as pltpu
from jax.experimental.pallas import tpu_sc as plsc

N = 10000
NP = 10240
E = 320000
EP = 327680
NC, NS = 2, 16
NW = NC * NS
EPW = EP // NW
B = 128
NB = EPW // B
RPT = NP // NS
C = 64
DW = 8

_mesh = plsc.VectorSubcoreMesh(core_axis_name="c", subcore_axis_name="s")
_sc_params = pltpu.CompilerParams(use_tc_tiling_on_sc=False)


@functools.partial(
    pl.kernel,
    out_type=jax.ShapeDtypeStruct((NC * NP, DW), jnp.float32),
    mesh=_mesh,
    scratch_types=[
        pltpu.VMEM((NB, B), jnp.int32),
        pltpu.VMEM((B, DW), jnp.float32),
        pltpu.VMEM_SHARED((NP, DW), jnp.float32),
    ],
    compiler_params=_sc_params,
)
def _deg_kernel(dst3, zeros_d, ones_d, degp, idx_d, ones_v, accum):
    c = lax.axis_index("c")
    s = lax.axis_index("s")
    wid = s * NC + c
    pltpu.sync_copy(zeros_d, accum.at[pl.ds(s * RPT, RPT)])
    pltpu.sync_copy(dst3.at[wid], idx_d)
    pltpu.sync_copy(ones_d, ones_v)
    plsc.subcore_barrier()

    def step(j, carry):
        pltpu.sync_copy(ones_v, accum.at[idx_d.at[j]], add=True)
        return carry

    lax.fori_loop(0, NB, step, 0)
    plsc.subcore_barrier()
    pltpu.sync_copy(accum.at[pl.ds(s * RPT, RPT)],
                    degp.at[pl.ds(c * NP + s * RPT, RPT)])


@functools.partial(
    pl.kernel,
    out_type=jax.ShapeDtypeStruct((NC * NP, C), jnp.float32),
    mesh=_mesh,
    scratch_types=[
        pltpu.VMEM((NB, B), jnp.int32),
        pltpu.VMEM((NB, B), jnp.int32),
        pltpu.VMEM((B, C), jnp.float32),
        pltpu.VMEM((B, C), jnp.float32),
        pltpu.VMEM_SHARED((NP, C), jnp.float32),
        pltpu.SemaphoreType.DMA,
        pltpu.SemaphoreType.DMA,
    ],
    compiler_params=_sc_params,
)
def _agg_kernel(hs, src3, dst3, zeros_c, out,
                idx_s, idx_d, rows0, rows1, accum, sem0, sem1):
    c = lax.axis_index("c")
    s = lax.axis_index("s")
    wid = s * NC + c
    pltpu.sync_copy(zeros_c, accum.at[pl.ds(s * RPT, RPT)])
    pltpu.sync_copy(src3.at[wid], idx_s)
    pltpu.sync_copy(dst3.at[wid], idx_d)
    plsc.subcore_barrier()

    pltpu.async_copy(hs.at[idx_s.at[0]], rows0, sem0)

    def step2(k, carry):
        j0 = 2 * k
        j1 = j0 + 1
        pltpu.make_async_copy(hs.at[idx_s.at[j0]], rows0, sem0).wait()
        pltpu.async_copy(hs.at[idx_s.at[j1]], rows1, sem1)
        pltpu.sync_copy(rows0, accum.at[idx_d.at[j0]], add=True)
        pltpu.make_async_copy(hs.at[idx_s.at[j1]], rows1, sem1).wait()

        @pl.when(k + 1 < NB // 2)
        def _():
            pltpu.async_copy(hs.at[idx_s.at[j0 + 2]], rows0, sem0)

        pltpu.sync_copy(rows1, accum.at[idx_d.at[j1]], add=True)
        return carry

    lax.fori_loop(0, NB // 2, step2, 0)
    plsc.subcore_barrier()
    pltpu.sync_copy(accum.at[pl.ds(s * RPT, RPT)],
                    out.at[pl.ds(c * NP + s * RPT, RPT)])


R = 1024
G = NP // R


def _dis_of(degp_ref):
    deg = degp_ref[0, :, 0:1] + degp_ref[1, :, 0:1] + 1.0
    return lax.rsqrt(deg)


def _mm1_body(degp_ref, x_ref, w1_ref, hs_ref):
    dis = _dis_of(degp_ref)
    h = jnp.dot(x_ref[...], w1_ref[...], preferred_element_type=jnp.float32)
    hs_ref[...] = h * dis


def _mid_body(degp_ref, p0_ref, p1_ref, hs_ref, b1_ref, u_ref):
    dis = _dis_of(degp_ref)
    agg = (p0_ref[...] + p1_ref[...] + hs_ref[...]) * dis + b1_ref[0:1, :]
    u = jnp.maximum(agg, 0.0) * dis
    row = (lax.broadcasted_iota(jnp.int32, (R, 1), 0)
           + pl.program_id(0) * R)
    u_ref[...] = jnp.where(row < N, u, 0.0)


def _mm2_body(degp_ref, q0_ref, q1_ref, u_ref, w2_ref, b2_ref, o_ref):
    dis = _dis_of(degp_ref)
    g = (q0_ref[...] + q1_ref[...] + u_ref[...]) * dis
    o_ref[...] = (jnp.dot(g, w2_ref[...], preferred_element_type=jnp.float32)
                  + b2_ref[0:1, :])


_degp_spec = pl.BlockSpec((NC, R, DW), lambda i: (0, i, 0))


_mm1 = pl.pallas_call(
    _mm1_body,
    grid=(G,),
    in_specs=[
        _degp_spec,
        pl.BlockSpec((R, 128), lambda i: (i, 0)),
        pl.BlockSpec((128, C), lambda i: (0, 0)),
    ],
    out_specs=pl.BlockSpec((R, C), lambda i: (i, 0)),
    out_shape=jax.ShapeDtypeStruct((NP, C), jnp.float32),
)

_mid = pl.pallas_call(
    _mid_body,
    grid=(G,),
    in_specs=[
        _degp_spec,
        pl.BlockSpec((R, C), lambda i: (i, 0)),
        pl.BlockSpec((R, C), lambda i: (G + i, 0)),
        pl.BlockSpec((R, C), lambda i: (i, 0)),
        pl.BlockSpec((8, C), lambda i: (0, 0)),
    ],
    out_specs=pl.BlockSpec((R, C), lambda i: (i, 0)),
    out_shape=jax.ShapeDtypeStruct((NP, C), jnp.float32),
)

_mm2 = pl.pallas_call(
    _mm2_body,
    grid=(G,),
    in_specs=[
        _degp_spec,
        pl.BlockSpec((R, C), lambda i: (i, 0)),
        pl.BlockSpec((R, C), lambda i: (G + i, 0)),
        pl.BlockSpec((R, C), lambda i: (i, 0)),
        pl.BlockSpec((C, 128), lambda i: (0, 0)),
        pl.BlockSpec((8, 128), lambda i: (0, 0)),
    ],
    out_specs=pl.BlockSpec((R, 128), lambda i: (i, 0)),
    out_shape=jax.ShapeDtypeStruct((NP, 128), jnp.float32),
)


def kernel(x, edge_index, W1, b1, W2, b2):
    ei = edge_index.astype(jnp.int32)
    pad = jnp.full((EP - E,), NP - 1, jnp.int32)
    src3 = jnp.concatenate([ei[0], pad]).reshape(NW, NB, B)
    dst3 = jnp.concatenate([ei[1], pad]).reshape(NW, NB, B)
    zeros_d = jnp.zeros((RPT, DW), jnp.float32)
    ones_d = jnp.ones((B, DW), jnp.float32)
    zeros_c = jnp.zeros((RPT, C), jnp.float32)
    xp = jnp.concatenate([x, jnp.zeros((NP - N, 128), jnp.float32)], axis=0)

    degp = _deg_kernel(dst3, zeros_d, ones_d).reshape(NC, NP, DW)
    hs1 = _mm1(degp, xp, W1)
    p = _agg_kernel(hs1, src3, dst3, zeros_c)
    u = _mid(degp, p, p, hs1, jnp.broadcast_to(b1.reshape(1, C), (8, C)))
    q = _agg_kernel(u, src3, dst3, zeros_c)
    out = _mm2(degp, q, q, u, W2,
               jnp.broadcast_to(b2.reshape(1, 128), (8, 128)))
    return out[:N]


# speedup vs baseline: 2.1364x; 2.1364x over previous
import functools

import jax
import jax.numpy as jnp
from jax import lax
from jax.experimental import pallas as pl
from jax.experimental.pallas import tpu as pltpu
from jax.experimental.pallas import tpu_sc as plsc

N = 10000
NP = 10240
E = 320000
EP = 327680
NC, NS = 2, 16
NW = NC * NS
EPW = EP // NW
B = 128
NB = EPW // B
RPT = NP // NS
C = 64
DW = 8

_mesh = plsc.VectorSubcoreMesh(core_axis_name="c", subcore_axis_name="s")
_sc_params = pltpu.CompilerParams(use_tc_tiling_on_sc=False)


@functools.partial(
    pl.kernel,
    out_type=jax.ShapeDtypeStruct((NC * NP, DW), jnp.float32),
    mesh=_mesh,
    scratch_types=[
        pltpu.VMEM((NB, B), jnp.int32),
        pltpu.VMEM((B, DW), jnp.float32),
        pltpu.VMEM_SHARED((NP, DW), jnp.float32),
    ],
    compiler_params=_sc_params,
)
def _deg_kernel(dst3, zeros_d, ones_d, degp, idx_d, ones_v, accum):
    c = lax.axis_index("c")
    s = lax.axis_index("s")
    wid = s * NC + c
    pltpu.sync_copy(zeros_d, accum.at[pl.ds(s * RPT, RPT)])
    pltpu.sync_copy(dst3.at[wid], idx_d)
    pltpu.sync_copy(ones_d, ones_v)
    plsc.subcore_barrier()

    def step(j, carry):
        pltpu.sync_copy(ones_v, accum.at[idx_d.at[j]], add=True)
        return carry

    lax.fori_loop(0, NB, step, 0)
    plsc.subcore_barrier()
    pltpu.sync_copy(accum.at[pl.ds(s * RPT, RPT)],
                    degp.at[pl.ds(c * NP + s * RPT, RPT)])


@functools.partial(
    pl.kernel,
    out_type=jax.ShapeDtypeStruct((NC * NP, C), jnp.float32),
    mesh=_mesh,
    scratch_types=[
        pltpu.VMEM((NB, B), jnp.int32),
        pltpu.VMEM((NB, B), jnp.int32),
        pltpu.VMEM((B, C), jnp.float32),
        pltpu.VMEM((B, C), jnp.float32),
        pltpu.VMEM_SHARED((NP, C), jnp.float32),
        pltpu.SemaphoreType.DMA,
        pltpu.SemaphoreType.DMA,
    ],
    compiler_params=_sc_params,
)
def _agg_kernel(hs, src3, dst3, zeros_c, out,
                idx_s, idx_d, rows0, rows1, accum, sem0, sem1):
    c = lax.axis_index("c")
    s = lax.axis_index("s")
    wid = s * NC + c
    pltpu.sync_copy(zeros_c, accum.at[pl.ds(s * RPT, RPT)])
    pltpu.sync_copy(src3.at[wid], idx_s)
    pltpu.sync_copy(dst3.at[wid], idx_d)
    plsc.subcore_barrier()

    pltpu.async_copy(hs.at[idx_s.at[0]], rows0, sem0)

    def step2(k, carry):
        j0 = 2 * k
        j1 = j0 + 1
        pltpu.make_async_copy(hs.at[idx_s.at[j0]], rows0, sem0).wait()
        pltpu.async_copy(hs.at[idx_s.at[j1]], rows1, sem1)
        pltpu.sync_copy(rows0, accum.at[idx_d.at[j0]], add=True)
        pltpu.make_async_copy(hs.at[idx_s.at[j1]], rows1, sem1).wait()

        @pl.when(k + 1 < NB // 2)
        def _():
            pltpu.async_copy(hs.at[idx_s.at[j0 + 2]], rows0, sem0)

        pltpu.sync_copy(rows1, accum.at[idx_d.at[j1]], add=True)
        return carry

    lax.fori_loop(0, NB // 2, step2, 0)
    plsc.subcore_barrier()
    pltpu.sync_copy(accum.at[pl.ds(s * RPT, RPT)],
                    out.at[pl.ds(c * NP + s * RPT, RPT)])


R = 1024
G = NP // R


def _dis_of(degp_ref):
    deg = degp_ref[0, :, 0:1] + degp_ref[1, :, 0:1] + 1.0
    return lax.rsqrt(deg)


def _mm1_body(degp_ref, x_ref, w1_ref, hs_ref):
    dis = _dis_of(degp_ref)
    h = jnp.dot(x_ref[...], w1_ref[...], preferred_element_type=jnp.float32)
    hs_ref[...] = h * dis


def _mid_body(degp_ref, p0_ref, p1_ref, hs_ref, b1_ref, u_ref):
    dis = _dis_of(degp_ref)
    agg = (p0_ref[...] + p1_ref[...] + hs_ref[...]) * dis + b1_ref[0:1, :]
    u = jnp.maximum(agg, 0.0) * dis
    row = (lax.broadcasted_iota(jnp.int32, (R, 1), 0)
           + pl.program_id(0) * R)
    u_ref[...] = jnp.where(row < N, u, 0.0)


def _mm2_body(degp_ref, q0_ref, q1_ref, u_ref, w2_ref, b2_ref, o_ref):
    dis = _dis_of(degp_ref)
    g = (q0_ref[...] + q1_ref[...] + u_ref[...]) * dis
    o_ref[...] = (jnp.dot(g, w2_ref[...], preferred_element_type=jnp.float32)
                  + b2_ref[0:1, :])


_degp_spec = pl.BlockSpec((NC, R, DW), lambda i: (0, i, 0))


_mm1 = pl.pallas_call(
    _mm1_body,
    grid=(G,),
    in_specs=[
        _degp_spec,
        pl.BlockSpec((R, 128), lambda i: (i, 0)),
        pl.BlockSpec((128, C), lambda i: (0, 0)),
    ],
    out_specs=pl.BlockSpec((R, C), lambda i: (i, 0)),
    out_shape=jax.ShapeDtypeStruct((NP, C), jnp.float32),
)

_mid = pl.pallas_call(
    _mid_body,
    grid=(G,),
    in_specs=[
        _degp_spec,
        pl.BlockSpec((R, C), lambda i: (i, 0)),
        pl.BlockSpec((R, C), lambda i: (G + i, 0)),
        pl.BlockSpec((R, C), lambda i: (i, 0)),
        pl.BlockSpec((8, C), lambda i: (0, 0)),
    ],
    out_specs=pl.BlockSpec((R, C), lambda i: (i, 0)),
    out_shape=jax.ShapeDtypeStruct((NP, C), jnp.float32),
)

_mm2 = pl.pallas_call(
    _mm2_body,
    grid=(G,),
    in_specs=[
        _degp_spec,
        pl.BlockSpec((R, C), lambda i: (i, 0)),
        pl.BlockSpec((R, C), lambda i: (G + i, 0)),
        pl.BlockSpec((R, C), lambda i: (i, 0)),
        pl.BlockSpec((C, 128), lambda i: (0, 0)),
        pl.BlockSpec((8, 128), lambda i: (0, 0)),
    ],
    out_specs=pl.BlockSpec((R, 128), lambda i: (i, 0)),
    out_shape=jax.ShapeDtypeStruct((NP, 128), jnp.float32),
)


def kernel(x, edge_index, W1, b1, W2, b2):
    ei = edge_index.astype(jnp.int32)
    pad = N + (jnp.arange(EP - E, dtype=jnp.int32) % (NP - N))
    src3 = jnp.concatenate([ei[0], pad]).reshape(NW, NB, B)
    dst3 = jnp.concatenate([ei[1], pad]).reshape(NW, NB, B)
    zeros_d = jnp.zeros((RPT, DW), jnp.float32)
    ones_d = jnp.ones((B, DW), jnp.float32)
    zeros_c = jnp.zeros((RPT, C), jnp.float32)
    xp = jnp.concatenate([x, jnp.zeros((NP - N, 128), jnp.float32)], axis=0)

    degp = _deg_kernel(dst3, zeros_d, ones_d).reshape(NC, NP, DW)
    hs1 = _mm1(degp, xp, W1)
    p = _agg_kernel(hs1, src3, dst3, zeros_c)
    u = _mid(degp, p, p, hs1, jnp.broadcast_to(b1.reshape(1, C), (8, C)))
    q = _agg_kernel(u, src3, dst3, zeros_c)
    out = _mm2(degp, q, q, u, W2,
               jnp.broadcast_to(b2.reshape(1, 128), (8, 128)))
    return out[:N]


# speedup vs baseline: 2.8067x; 1.3137x over previous
import functools

import jax
import jax.numpy as jnp
from jax import lax
from jax.experimental import pallas as pl
from jax.experimental.pallas import tpu as pltpu
from jax.experimental.pallas import tpu_sc as plsc

N = 10000
NP = 10240
E = 320000
EP = 327680
NC, NS = 2, 16
NW = NC * NS
EPW = EP // NW
B = 128
NB = EPW // B
NBUF = 4
RPT = NP // NS
C = 64
DW = 8

_mesh = plsc.VectorSubcoreMesh(core_axis_name="c", subcore_axis_name="s")
_sc_params = pltpu.CompilerParams(use_tc_tiling_on_sc=False)


@functools.partial(
    pl.kernel,
    out_type=jax.ShapeDtypeStruct((NC * NP, DW), jnp.float32),
    mesh=_mesh,
    scratch_types=[
        pltpu.VMEM((NB, B), jnp.int32),
        pltpu.VMEM((B, DW), jnp.float32),
        pltpu.VMEM_SHARED((NP, DW), jnp.float32),
    ],
    compiler_params=_sc_params,
)
def _deg_kernel(dst3, zeros_d, ones_d, degp, idx_d, ones_v, accum):
    c = lax.axis_index("c")
    s = lax.axis_index("s")
    wid = s * NC + c
    pltpu.sync_copy(zeros_d, accum.at[pl.ds(s * RPT, RPT)])
    pltpu.sync_copy(dst3.at[wid], idx_d)
    pltpu.sync_copy(ones_d, ones_v)
    plsc.subcore_barrier()

    def step(j, carry):
        pltpu.sync_copy(ones_v, accum.at[idx_d.at[j]], add=True)
        return carry

    lax.fori_loop(0, NB, step, 0)
    plsc.subcore_barrier()
    pltpu.sync_copy(accum.at[pl.ds(s * RPT, RPT)],
                    degp.at[pl.ds(c * NP + s * RPT, RPT)])


@functools.partial(
    pl.kernel,
    out_type=jax.ShapeDtypeStruct((NC * NP, C), jnp.float32),
    mesh=_mesh,
    scratch_types=[
        pltpu.VMEM((NB, B), jnp.int32),
        pltpu.VMEM((NB, B), jnp.int32),
        pltpu.VMEM((B, C), jnp.float32),
        pltpu.VMEM((B, C), jnp.float32),
        pltpu.VMEM((B, C), jnp.float32),
        pltpu.VMEM((B, C), jnp.float32),
        pltpu.VMEM_SHARED((NP, C), jnp.float32),
        pltpu.SemaphoreType.DMA,
        pltpu.SemaphoreType.DMA,
        pltpu.SemaphoreType.DMA,
        pltpu.SemaphoreType.DMA,
    ],
    compiler_params=_sc_params,
)
def _agg_kernel(hs, src3, dst3, zeros_c, out,
                idx_s, idx_d, rows0, rows1, rows2, rows3, accum,
                sem0, sem1, sem2, sem3):
    c = lax.axis_index("c")
    s = lax.axis_index("s")
    wid = s * NC + c
    rowsb = (rows0, rows1, rows2, rows3)
    semb = (sem0, sem1, sem2, sem3)
    pltpu.sync_copy(zeros_c, accum.at[pl.ds(s * RPT, RPT)])
    pltpu.sync_copy(src3.at[wid], idx_s)
    pltpu.sync_copy(dst3.at[wid], idx_d)
    plsc.subcore_barrier()

    for b in range(NBUF):
        pltpu.async_copy(hs.at[idx_s.at[b]], rowsb[b], semb[b])

    def stepg(g, carry):
        j0 = g * NBUF
        for b in range(NBUF):
            j = j0 + b
            pltpu.make_async_copy(hs.at[idx_s.at[j]], rowsb[b],
                                  semb[b]).wait()
            pltpu.sync_copy(rowsb[b], accum.at[idx_d.at[j]], add=True)

            @pl.when(j + NBUF < NB)
            def _(b=b, j=j):
                pltpu.async_copy(hs.at[idx_s.at[j + NBUF]], rowsb[b],
                                 semb[b])

        return carry

    lax.fori_loop(0, NB // NBUF, stepg, 0)
    plsc.subcore_barrier()
    pltpu.sync_copy(accum.at[pl.ds(s * RPT, RPT)],
                    out.at[pl.ds(c * NP + s * RPT, RPT)])


R = 1024
G = NP // R


def _dis_of(degp_ref):
    deg = degp_ref[0, :, 0:1] + degp_ref[1, :, 0:1] + 1.0
    return lax.rsqrt(deg)


def _mm1_body(degp_ref, x_ref, w1_ref, hs_ref):
    dis = _dis_of(degp_ref)
    h = jnp.dot(x_ref[...], w1_ref[...], preferred_element_type=jnp.float32)
    hs_ref[...] = h * dis


def _mid_body(degp_ref, p0_ref, p1_ref, hs_ref, b1_ref, u_ref):
    dis = _dis_of(degp_ref)
    agg = (p0_ref[...] + p1_ref[...] + hs_ref[...]) * dis + b1_ref[0:1, :]
    u = jnp.maximum(agg, 0.0) * dis
    row = (lax.broadcasted_iota(jnp.int32, (R, 1), 0)
           + pl.program_id(0) * R)
    u_ref[...] = jnp.where(row < N, u, 0.0)


def _mm2_body(degp_ref, q0_ref, q1_ref, u_ref, w2_ref, b2_ref, o_ref):
    dis = _dis_of(degp_ref)
    g = (q0_ref[...] + q1_ref[...] + u_ref[...]) * dis
    o_ref[...] = (jnp.dot(g, w2_ref[...], preferred_element_type=jnp.float32)
                  + b2_ref[0:1, :])


_degp_spec = pl.BlockSpec((NC, R, DW), lambda i: (0, i, 0))


_mm1 = pl.pallas_call(
    _mm1_body,
    grid=(G,),
    in_specs=[
        _degp_spec,
        pl.BlockSpec((R, 128), lambda i: (i, 0)),
        pl.BlockSpec((128, C), lambda i: (0, 0)),
    ],
    out_specs=pl.BlockSpec((R, C), lambda i: (i, 0)),
    out_shape=jax.ShapeDtypeStruct((NP, C), jnp.float32),
)

_mid = pl.pallas_call(
    _mid_body,
    grid=(G,),
    in_specs=[
        _degp_spec,
        pl.BlockSpec((R, C), lambda i: (i, 0)),
        pl.BlockSpec((R, C), lambda i: (G + i, 0)),
        pl.BlockSpec((R, C), lambda i: (i, 0)),
        pl.BlockSpec((8, C), lambda i: (0, 0)),
    ],
    out_specs=pl.BlockSpec((R, C), lambda i: (i, 0)),
    out_shape=jax.ShapeDtypeStruct((NP, C), jnp.float32),
)

_mm2 = pl.pallas_call(
    _mm2_body,
    grid=(G,),
    in_specs=[
        _degp_spec,
        pl.BlockSpec((R, C), lambda i: (i, 0)),
        pl.BlockSpec((R, C), lambda i: (G + i, 0)),
        pl.BlockSpec((R, C), lambda i: (i, 0)),
        pl.BlockSpec((C, 128), lambda i: (0, 0)),
        pl.BlockSpec((8, 128), lambda i: (0, 0)),
    ],
    out_specs=pl.BlockSpec((R, 128), lambda i: (i, 0)),
    out_shape=jax.ShapeDtypeStruct((NP, 128), jnp.float32),
)


def kernel(x, edge_index, W1, b1, W2, b2):
    ei = edge_index.astype(jnp.int32)
    pad = N + (jnp.arange(EP - E, dtype=jnp.int32) % (NP - N))
    src3 = jnp.concatenate([ei[0], pad]).reshape(NW, NB, B)
    dst3 = jnp.concatenate([ei[1], pad]).reshape(NW, NB, B)
    zeros_d = jnp.zeros((RPT, DW), jnp.float32)
    ones_d = jnp.ones((B, DW), jnp.float32)
    zeros_c = jnp.zeros((RPT, C), jnp.float32)
    xp = jnp.concatenate([x, jnp.zeros((NP - N, 128), jnp.float32)], axis=0)

    degp = _deg_kernel(dst3, zeros_d, ones_d).reshape(NC, NP, DW)
    hs1 = _mm1(degp, xp, W1)
    p = _agg_kernel(hs1, src3, dst3, zeros_c)
    u = _mid(degp, p, p, hs1, jnp.broadcast_to(b1.reshape(1, C), (8, C)))
    q = _agg_kernel(u, src3, dst3, zeros_c)
    out = _mm2(degp, q, q, u, W2,
               jnp.broadcast_to(b2.reshape(1, 128), (8, 128)))
    return out[:N]


# speedup vs baseline: 2.9067x; 1.0356x over previous
import functools

import jax
import jax.numpy as jnp
from jax import lax
from jax.experimental import pallas as pl
from jax.experimental.pallas import tpu as pltpu
from jax.experimental.pallas import tpu_sc as plsc

N = 10000
NP = 10240
E = 320000
NC, NS = 2, 16
NW = NC * NS
EPW = E // NW
B = 80
NB = EPW // B
NBUF = 5
RPT = NP // NS
C = 64
DW = 8

_mesh = plsc.VectorSubcoreMesh(core_axis_name="c", subcore_axis_name="s")
_sc_params = pltpu.CompilerParams(use_tc_tiling_on_sc=False)


@functools.partial(
    pl.kernel,
    out_type=(jax.ShapeDtypeStruct((NP, DW), jnp.float32),
              jax.ShapeDtypeStruct((NP, DW), jnp.float32)),
    mesh=_mesh,
    scratch_types=[
        pltpu.VMEM((NB, B), jnp.int32),
        pltpu.VMEM((B, DW), jnp.float32),
        pltpu.VMEM_SHARED((NP, DW), jnp.float32),
    ],
    compiler_params=_sc_params,
)
def _deg_kernel(dst3, zeros_d, ones_d, deg0, deg1, idx_d, ones_v, accum):
    c = lax.axis_index("c")
    s = lax.axis_index("s")
    wid = s * NC + c
    pltpu.sync_copy(zeros_d, accum.at[pl.ds(s * RPT, RPT)])
    pltpu.sync_copy(dst3.at[wid], idx_d)
    pltpu.sync_copy(ones_d, ones_v)
    plsc.subcore_barrier()

    def step(j, carry):
        pltpu.sync_copy(ones_v, accum.at[idx_d.at[j]], add=True)
        return carry

    lax.fori_loop(0, NB, step, 0)
    plsc.subcore_barrier()

    @pl.when(c == 0)
    def _():
        pltpu.sync_copy(accum.at[pl.ds(s * RPT, RPT)],
                        deg0.at[pl.ds(s * RPT, RPT)])

    @pl.when(c == 1)
    def _():
        pltpu.sync_copy(accum.at[pl.ds(s * RPT, RPT)],
                        deg1.at[pl.ds(s * RPT, RPT)])


@functools.partial(
    pl.kernel,
    out_type=(jax.ShapeDtypeStruct((NP, C), jnp.float32),
              jax.ShapeDtypeStruct((NP, C), jnp.float32)),
    mesh=_mesh,
    scratch_types=[
        pltpu.VMEM((NB, B), jnp.int32),
        pltpu.VMEM((NB, B), jnp.int32),
        pltpu.VMEM((B, C), jnp.float32),
        pltpu.VMEM((B, C), jnp.float32),
        pltpu.VMEM((B, C), jnp.float32),
        pltpu.VMEM((B, C), jnp.float32),
        pltpu.VMEM((B, C), jnp.float32),
        pltpu.VMEM_SHARED((NP, C), jnp.float32),
        pltpu.SemaphoreType.DMA,
        pltpu.SemaphoreType.DMA,
        pltpu.SemaphoreType.DMA,
        pltpu.SemaphoreType.DMA,
        pltpu.SemaphoreType.DMA,
    ],
    compiler_params=_sc_params,
)
def _agg_kernel(hs, src3, dst3, zeros_c, out0, out1,
                idx_s, idx_d, rows0, rows1, rows2, rows3, rows4, accum,
                sem0, sem1, sem2, sem3, sem4):
    c = lax.axis_index("c")
    s = lax.axis_index("s")
    wid = s * NC + c
    rowsb = (rows0, rows1, rows2, rows3, rows4)
    semb = (sem0, sem1, sem2, sem3, sem4)
    pltpu.sync_copy(zeros_c, accum.at[pl.ds(s * RPT, RPT)])
    pltpu.sync_copy(src3.at[wid], idx_s)
    pltpu.sync_copy(dst3.at[wid], idx_d)
    plsc.subcore_barrier()

    for b in range(NBUF):
        pltpu.async_copy(hs.at[idx_s.at[b]], rowsb[b], semb[b])

    def stepg(g, carry):
        j0 = g * NBUF
        for b in range(NBUF):
            j = j0 + b
            pltpu.make_async_copy(hs.at[idx_s.at[j]], rowsb[b],
                                  semb[b]).wait()
            pltpu.sync_copy(rowsb[b], accum.at[idx_d.at[j]], add=True)

            @pl.when(j + NBUF < NB)
            def _(b=b, j=j):
                pltpu.async_copy(hs.at[idx_s.at[j + NBUF]], rowsb[b],
                                 semb[b])

        return carry

    lax.fori_loop(0, NB // NBUF, stepg, 0)
    plsc.subcore_barrier()

    @pl.when(c == 0)
    def _():
        pltpu.sync_copy(accum.at[pl.ds(s * RPT, RPT)],
                        out0.at[pl.ds(s * RPT, RPT)])

    @pl.when(c == 1)
    def _():
        pltpu.sync_copy(accum.at[pl.ds(s * RPT, RPT)],
                        out1.at[pl.ds(s * RPT, RPT)])


R = 1000
G = N // R


def _dis_of(d0_ref, d1_ref):
    deg = d0_ref[:, 0:1] + d1_ref[:, 0:1] + 1.0
    return lax.rsqrt(deg)


def _mm1_body(d0_ref, d1_ref, x_ref, w1_ref, hs_ref):
    dis = _dis_of(d0_ref, d1_ref)
    h = jnp.dot(x_ref[...], w1_ref[...], preferred_element_type=jnp.float32)
    hs_ref[...] = h * dis


def _mid_body(d0_ref, d1_ref, p0_ref, p1_ref, hs_ref, b1_ref, u_ref):
    dis = _dis_of(d0_ref, d1_ref)
    agg = (p0_ref[...] + p1_ref[...] + hs_ref[...]) * dis + b1_ref[0:1, :]
    u_ref[...] = jnp.maximum(agg, 0.0) * dis


def _mm2_body(d0_ref, d1_ref, q0_ref, q1_ref, u_ref, w2_ref, b2_ref, o_ref):
    dis = _dis_of(d0_ref, d1_ref)
    g = (q0_ref[...] + q1_ref[...] + u_ref[...]) * dis
    o_ref[...] = (jnp.dot(g, w2_ref[...], preferred_element_type=jnp.float32)
                  + b2_ref[0:1, :])


_row_spec_d = pl.BlockSpec((R, DW), lambda i: (i, 0))
_row_spec_c = pl.BlockSpec((R, C), lambda i: (i, 0))


_mm1 = pl.pallas_call(
    _mm1_body,
    grid=(G,),
    in_specs=[
        _row_spec_d,
        _row_spec_d,
        pl.BlockSpec((R, 128), lambda i: (i, 0)),
        pl.BlockSpec((128, C), lambda i: (0, 0)),
    ],
    out_specs=_row_spec_c,
    out_shape=jax.ShapeDtypeStruct((N, C), jnp.float32),
)

_mid = pl.pallas_call(
    _mid_body,
    grid=(G,),
    in_specs=[
        _row_spec_d,
        _row_spec_d,
        _row_spec_c,
        _row_spec_c,
        _row_spec_c,
        pl.BlockSpec((8, C), lambda i: (0, 0)),
    ],
    out_specs=_row_spec_c,
    out_shape=jax.ShapeDtypeStruct((N, C), jnp.float32),
)

_mm2 = pl.pallas_call(
    _mm2_body,
    grid=(G,),
    in_specs=[
        _row_spec_d,
        _row_spec_d,
        _row_spec_c,
        _row_spec_c,
        _row_spec_c,
        pl.BlockSpec((C, 128), lambda i: (0, 0)),
        pl.BlockSpec((8, 128), lambda i: (0, 0)),
    ],
    out_specs=pl.BlockSpec((R, 128), lambda i: (i, 0)),
    out_shape=jax.ShapeDtypeStruct((N, 128), jnp.float32),
)


def kernel(x, edge_index, W1, b1, W2, b2):
    ei = edge_index.astype(jnp.int32)
    src3 = ei[0].reshape(NW, NB, B)
    dst3 = ei[1].reshape(NW, NB, B)
    zeros_d = jnp.zeros((RPT, DW), jnp.float32)
    ones_d = jnp.ones((B, DW), jnp.float32)
    zeros_c = jnp.zeros((RPT, C), jnp.float32)

    deg0, deg1 = _deg_kernel(dst3, zeros_d, ones_d)
    hs1 = _mm1(deg0, deg1, x, W1)
    p0, p1 = _agg_kernel(hs1, src3, dst3, zeros_c)
    u = _mid(deg0, deg1, p0, p1, hs1, jnp.broadcast_to(b1.reshape(1, C), (8, C)))
    q0, q1 = _agg_kernel(u, src3, dst3, zeros_c)
    return _mm2(deg0, deg1, q0, q1, u, W2,
                jnp.broadcast_to(b2.reshape(1, 128), (8, 128)))
